# Initial kernel scaffold; baseline (speedup 1.0000x reference)
import jax
import jax.numpy as jnp
from jax import lax
from jax.experimental import pallas as pl
from jax.experimental.pallas import tpu as pltpu
from jax.experimental.pallas import tpu_sc as plsc

N_CORES = 2
N_SUBCORES = 16
N_WORKERS = N_CORES * N_SUBCORES
LANES = 16

M_TOTAL = 200000
K_IDX = 3
D_FEAT = 128
W = 64
SLAB = 6272
TAIL = M_TOTAL - (N_WORKERS - 1) * SLAB
NWIN_FULL = SLAB // W
NWIN_TAIL = TAIL // W


def _sum_body(x_hbm, idx_hbm, out_hbm, idx_v, b0, b1, b2, sem):
    cid = lax.axis_index("c")
    sid = lax.axis_index("s")
    wid = sid * N_CORES + cid
    base = wid * SLAB
    nrows = jnp.where(wid == N_WORKERS - 1, TAIL, SLAB)
    nwin = jnp.where(wid == N_WORKERS - 1, NWIN_TAIL, NWIN_FULL)

    for k in range(K_IDX):
        pltpu.sync_copy(idx_hbm.at[k, pl.ds(base, TAIL)], idx_v.at[k, pl.ds(0, TAIL)])
    rest = SLAB - TAIL

    @pl.when(wid < N_WORKERS - 1)
    def _():
        for k in range(K_IDX):
            pltpu.sync_copy(
                idx_hbm.at[k, pl.ds(base + TAIL, rest)],
                idx_v.at[k, pl.ds(TAIL, rest)],
            )

    del nrows

    @pl.loop(0, nwin)
    def _(i):
        off = i * W
        cps = [
            pltpu.async_copy(
                x_hbm.at[idx_v.at[k, pl.ds(off, W)]], [b0, b1, b2][k], sem
            )
            for k in range(K_IDX)
        ]
        for cp in cps:
            cp.wait()

        @pl.loop(0, W)
        def _(j):
            @pl.loop(0, D_FEAT, step=LANES)
            def _(c):
                s = (j, pl.ds(c, LANES))
                b0[s] = b0[s] + b1[s] + b2[s]

        pltpu.sync_copy(b0, out_hbm.at[pl.ds(base + off, W)])


def _build_kernel():
    mesh = plsc.VectorSubcoreMesh(core_axis_name="c", subcore_axis_name="s")
    return pl.kernel(
        _sum_body,
        out_type=jax.ShapeDtypeStruct((M_TOTAL, D_FEAT), jnp.float32),
        mesh=mesh,
        scratch_types=[
            pltpu.VMEM((K_IDX, SLAB), jnp.int32),
            pltpu.VMEM((W, D_FEAT), jnp.float32),
            pltpu.VMEM((W, D_FEAT), jnp.float32),
            pltpu.VMEM((W, D_FEAT), jnp.float32),
            pltpu.SemaphoreType.DMA,
        ],
    )


@jax.jit
def kernel(x, indices):
    idx_t = indices.astype(jnp.int32).T
    return _build_kernel()(x, idx_t)


# speedup vs baseline: 10.7372x; 10.7372x over previous
import jax
import jax.numpy as jnp
from jax import lax
from jax.experimental import pallas as pl
from jax.experimental.pallas import tpu as pltpu
from jax.experimental.pallas import tpu_sc as plsc

N_CORES = 2
N_SUBCORES = 16
N_WORKERS = N_CORES * N_SUBCORES
LANES = 16

M_TOTAL = 200000
K_IDX = 3
D_FEAT = 128
W = 64
SLAB = 6272
TAIL = M_TOTAL - (N_WORKERS - 1) * SLAB
NWIN_FULL = SLAB // W
NWIN_TAIL = TAIL // W


def _sum_body(x_hbm, i0_hbm, i1_hbm, i2_hbm, out_hbm, idx0_v, idx1_v, idx2_v, b0, b1, b2, sem):
    cid = lax.axis_index("c")
    sid = lax.axis_index("s")
    wid = sid * N_CORES + cid
    base = wid * SLAB
    nrows = jnp.where(wid == N_WORKERS - 1, TAIL, SLAB)
    nwin = jnp.where(wid == N_WORKERS - 1, NWIN_TAIL, NWIN_FULL)

    del nrows

    idx_bufs = (idx0_v, idx1_v, idx2_v)
    for k, src_hbm in enumerate((i0_hbm, i1_hbm, i2_hbm)):
        pltpu.sync_copy(src_hbm.at[pl.ds(base, SLAB)], idx_bufs[k])

    @pl.loop(0, nwin)
    def _(i):
        off = i * W
        cps = [
            pltpu.async_copy(
                x_hbm.at[idx_bufs[k].at[pl.ds(off, W)]], [b0, b1, b2][k], sem
            )
            for k in range(K_IDX)
        ]
        for cp in cps:
            cp.wait()

        @pl.loop(0, W)
        def _(j):
            @pl.loop(0, D_FEAT, step=LANES)
            def _(c):
                s = (j, pl.ds(c, LANES))
                b0[s] = b0[s] + b1[s] + b2[s]

        pltpu.sync_copy(b0, out_hbm.at[pl.ds(base + off, W)])


def _build_kernel():
    mesh = plsc.VectorSubcoreMesh(core_axis_name="c", subcore_axis_name="s")
    return pl.kernel(
        _sum_body,
        out_type=jax.ShapeDtypeStruct((M_TOTAL, D_FEAT), jnp.float32),
        mesh=mesh,
        scratch_types=[
            pltpu.VMEM((SLAB,), jnp.int32),
            pltpu.VMEM((SLAB,), jnp.int32),
            pltpu.VMEM((SLAB,), jnp.int32),
            pltpu.VMEM((W, D_FEAT), jnp.float32),
            pltpu.VMEM((W, D_FEAT), jnp.float32),
            pltpu.VMEM((W, D_FEAT), jnp.float32),
            pltpu.SemaphoreType.DMA,
        ],
    )


@jax.jit
def kernel(x, indices):
    idx = indices.astype(jnp.int32)
    pad = N_WORKERS * SLAB - M_TOTAL
    idx = jnp.pad(idx, ((0, pad), (0, 0)))
    return _build_kernel()(x, idx[:, 0], idx[:, 1], idx[:, 2])


# speedup vs baseline: 18.3263x; 1.7068x over previous
import jax
import jax.numpy as jnp
from jax import lax
from jax.experimental import pallas as pl
from jax.experimental.pallas import tpu as pltpu
from jax.experimental.pallas import tpu_sc as plsc

N_CORES = 2
N_SUBCORES = 16
N_WORKERS = N_CORES * N_SUBCORES
LANES = 16

M_TOTAL = 200000
K_IDX = 3
D_FEAT = 128
W = 64
SLAB = 6272
TAIL = M_TOTAL - (N_WORKERS - 1) * SLAB
NWIN_FULL = SLAB // W
NWIN_TAIL = TAIL // W


def _sum_body(x_hbm, i0_hbm, i1_hbm, i2_hbm, out_hbm,
              idx0_v, idx1_v, idx2_v,
              a0, b0, c0, a1, b1, c1,
              sem_g0, sem_g1, sem_o0, sem_o1):
    cid = lax.axis_index("c")
    sid = lax.axis_index("s")
    wid = sid * N_CORES + cid
    base = wid * SLAB
    nwin = jnp.where(wid == N_WORKERS - 1, NWIN_TAIL, NWIN_FULL)

    idx_bufs = (idx0_v, idx1_v, idx2_v)
    for k, src_hbm in enumerate((i0_hbm, i1_hbm, i2_hbm)):
        pltpu.sync_copy(src_hbm.at[pl.ds(base, SLAB)], idx_bufs[k])

    sets = ((a0, b0, c0, sem_g0, sem_o0), (a1, b1, c1, sem_g1, sem_o1))

    def g_copies(j, s):
        off = j * W
        return [
            pltpu.make_async_copy(
                x_hbm.at[idx_bufs[k].at[pl.ds(off, W)]], sets[s][k], sets[s][3]
            )
            for k in range(K_IDX)
        ]

    def o_copy(j, s):
        return pltpu.make_async_copy(
            sets[s][0], out_hbm.at[pl.ds(base + j * W, W)], sets[s][4]
        )

    for cp in g_copies(0, 0):
        cp.start()

    def half(i, cur, nxt):
        j = i + 1

        @pl.when(jnp.logical_and(j >= 2, j < nwin))
        def _():
            o_copy(j - 2, nxt).wait()

        @pl.when(j < nwin)
        def _():
            for cp in g_copies(j, nxt):
                cp.start()

        @pl.when(i < nwin)
        def _():
            for cp in g_copies(i, cur):
                cp.wait()
            r0, r1, r2 = sets[cur][:3]

            @pl.loop(0, W)
            def _(jr):
                @pl.loop(0, D_FEAT, step=LANES, unroll=True)
                def _(c):
                    s = (jr, pl.ds(c, LANES))
                    r0[s] = r0[s] + r1[s] + r2[s]

            o_copy(i, cur).start()

    @pl.loop(0, NWIN_FULL, step=2)
    def _(i):
        half(i, 0, 1)
        half(i + 1, 1, 0)

    j0 = nwin - 2 + (nwin % 2)
    j1 = nwin - 1 - (nwin % 2)
    o_copy(j0, 0).wait()
    o_copy(j1, 1).wait()


def _build_kernel():
    mesh = plsc.VectorSubcoreMesh(core_axis_name="c", subcore_axis_name="s")
    return pl.kernel(
        _sum_body,
        out_type=jax.ShapeDtypeStruct((M_TOTAL, D_FEAT), jnp.float32),
        mesh=mesh,
        scratch_types=[
            pltpu.VMEM((SLAB,), jnp.int32),
            pltpu.VMEM((SLAB,), jnp.int32),
            pltpu.VMEM((SLAB,), jnp.int32),
            pltpu.VMEM((W, D_FEAT), jnp.float32),
            pltpu.VMEM((W, D_FEAT), jnp.float32),
            pltpu.VMEM((W, D_FEAT), jnp.float32),
            pltpu.VMEM((W, D_FEAT), jnp.float32),
            pltpu.VMEM((W, D_FEAT), jnp.float32),
            pltpu.VMEM((W, D_FEAT), jnp.float32),
            pltpu.SemaphoreType.DMA,
            pltpu.SemaphoreType.DMA,
            pltpu.SemaphoreType.DMA,
            pltpu.SemaphoreType.DMA,
        ],
    )


@jax.jit
def kernel(x, indices):
    idx = indices.astype(jnp.int32)
    pad = N_WORKERS * SLAB - M_TOTAL
    idx = jnp.pad(idx, ((0, pad), (0, 0)))
    return _build_kernel()(x, idx[:, 0], idx[:, 1], idx[:, 2])


# speedup vs baseline: 18.3501x; 1.0013x over previous
import jax
import jax.numpy as jnp
from jax import lax
from jax.experimental import pallas as pl
from jax.experimental.pallas import tpu as pltpu
from jax.experimental.pallas import tpu_sc as plsc

N_CORES = 2
N_SUBCORES = 16
N_WORKERS = N_CORES * N_SUBCORES
LANES = 16

M_TOTAL = 200000
K_IDX = 3
D_FEAT = 128
W = 64
SLAB = 6272
TAIL = M_TOTAL - (N_WORKERS - 1) * SLAB
NWIN_FULL = SLAB // W
NWIN_TAIL = TAIL // W


def _sum_body(x_hbm, i0_hbm, i1_hbm, i2_hbm, out_hbm,
              idx0_v, idx1_v, idx2_v,
              a0, b0, c0, a1, b1, c1,
              sem_g0, sem_g1, sem_o0, sem_o1):
    cid = lax.axis_index("c")
    sid = lax.axis_index("s")
    wid = sid * N_CORES + cid
    base = wid * SLAB
    nwin = jnp.where(wid == N_WORKERS - 1, NWIN_TAIL, NWIN_FULL)

    idx_bufs = (idx0_v, idx1_v, idx2_v)
    for k, src_hbm in enumerate((i0_hbm, i1_hbm, i2_hbm)):
        pltpu.sync_copy(src_hbm.at[pl.ds(base, SLAB)], idx_bufs[k])

    sets = ((a0, b0, c0, sem_g0, sem_o0), (a1, b1, c1, sem_g1, sem_o1))

    def g_copies(j, s):
        off = j * W
        return [
            pltpu.make_async_copy(
                x_hbm.at[idx_bufs[k].at[pl.ds(off, W)]], sets[s][k], sets[s][3]
            )
            for k in range(K_IDX)
        ]

    def o_copy(j, s):
        return pltpu.make_async_copy(
            sets[s][0], out_hbm.at[pl.ds(base + j * W, W)], sets[s][4]
        )

    for cp in g_copies(0, 0):
        cp.start()

    def half(i, cur, nxt):
        j = i + 1

        @pl.when(jnp.logical_and(j >= 2, j < nwin))
        def _():
            o_copy(j - 2, nxt).wait()

        @pl.when(j < nwin)
        def _():
            for cp in g_copies(j, nxt):
                cp.start()

        @pl.when(i < nwin)
        def _():
            for cp in g_copies(i, cur):
                cp.wait()
            r0, r1, r2 = sets[cur][:3]

            @pl.loop(0, W, unroll=2)
            def _(jr):
                @pl.loop(0, D_FEAT, step=LANES, unroll=True)
                def _(c):
                    s = (jr, pl.ds(c, LANES))
                    plsc.addupdate(r0.at[s], r1[s] + r2[s])

            o_copy(i, cur).start()

    @pl.loop(0, NWIN_FULL, step=2)
    def _(i):
        half(i, 0, 1)
        half(i + 1, 1, 0)

    j0 = nwin - 2 + (nwin % 2)
    j1 = nwin - 1 - (nwin % 2)
    o_copy(j0, 0).wait()
    o_copy(j1, 1).wait()


def _build_kernel():
    mesh = plsc.VectorSubcoreMesh(core_axis_name="c", subcore_axis_name="s")
    return pl.kernel(
        _sum_body,
        out_type=jax.ShapeDtypeStruct((M_TOTAL, D_FEAT), jnp.float32),
        mesh=mesh,
        scratch_types=[
            pltpu.VMEM((SLAB,), jnp.int32),
            pltpu.VMEM((SLAB,), jnp.int32),
            pltpu.VMEM((SLAB,), jnp.int32),
            pltpu.VMEM((W, D_FEAT), jnp.float32),
            pltpu.VMEM((W, D_FEAT), jnp.float32),
            pltpu.VMEM((W, D_FEAT), jnp.float32),
            pltpu.VMEM((W, D_FEAT), jnp.float32),
            pltpu.VMEM((W, D_FEAT), jnp.float32),
            pltpu.VMEM((W, D_FEAT), jnp.float32),
            pltpu.SemaphoreType.DMA,
            pltpu.SemaphoreType.DMA,
            pltpu.SemaphoreType.DMA,
            pltpu.SemaphoreType.DMA,
        ],
    )


@jax.jit
def kernel(x, indices):
    idx = indices.astype(jnp.int32)
    pad = N_WORKERS * SLAB - M_TOTAL
    idx = jnp.pad(idx, ((0, pad), (0, 0)))
    return _build_kernel()(x, idx[:, 0], idx[:, 1], idx[:, 2])
